# baseline (speedup 1.0000x reference)
import functools

import jax
import jax.numpy as jnp
from jax import lax
from jax.experimental import pallas as pl
from jax.experimental.pallas import tpu as pltpu
from jax.experimental.pallas import tpu_sc as plsc

_ROWS = 16384
_NC = 2
_ROWS_PER_C = _ROWS // _NC
_CH = 256
_NITER = _ROWS_PER_C // _CH
_NBUF = 3
_A = 1


def _scs_copy(x_hbm, o_hbm, buf, in_sems, out_sems):
    c = lax.axis_index("c")
    base = c * _ROWS_PER_C

    def in_copy(k, b):
        return pltpu.make_async_copy(
            x_hbm.at[pl.ds(base + k * _CH, _CH)], buf.at[b], in_sems.at[b]
        )

    def out_copy(k, b):
        return pltpu.make_async_copy(
            buf.at[b], o_hbm.at[pl.ds(base + k * _CH, _CH)], out_sems.at[b]
        )

    def body(g):
        for b in range(_NBUF):
            k = g * _NBUF + b

            @pl.when(k >= _NBUF)
            def _():
                out_copy(k - _NBUF, b).wait()

            in_copy(k, b).start()

            bb = (b - _A) % _NBUF

            @pl.when(k >= _A)
            def _():
                in_copy(k - _A, bb).wait()
                out_copy(k - _A, bb).start()

    pl.loop(0, _NITER // _NBUF)(body)
    for k in range(_NITER - _NITER % _NBUF, _NITER):
        b = k % _NBUF

        @pl.when(k >= _NBUF)
        def _():
            out_copy(k - _NBUF, b).wait()

        in_copy(k, b).start()
        bb = (k - _A) % _NBUF
        in_copy(k - _A, bb).wait()
        out_copy(k - _A, bb).start()
    for k in range(_NITER - _A, _NITER):
        in_copy(k, k % _NBUF).wait()
        out_copy(k, k % _NBUF).start()
    for k in range(_NITER - _NBUF, _NITER):
        out_copy(k, k % _NBUF).wait()


def kernel(inputs, memories):
    del memories
    B, T, d = inputs.shape
    x = inputs.reshape(B * T, d)
    mesh = plsc.ScalarSubcoreMesh(axis_name="c", num_cores=_NC)
    run = functools.partial(
        pl.kernel,
        mesh=mesh,
        out_type=jax.ShapeDtypeStruct((B * T, d), jnp.float32),
        scratch_types=[
            pltpu.VMEM_SHARED((_NBUF, _CH, d), jnp.float32),
            pltpu.SemaphoreType.DMA((_NBUF,)),
            pltpu.SemaphoreType.DMA((_NBUF,)),
        ],
    )(_scs_copy)
    return run(x).reshape(B, T, d)


# speedup vs baseline: 1.0517x; 1.0517x over previous
import functools

import jax
import jax.numpy as jnp
from jax import lax
from jax.experimental import pallas as pl
from jax.experimental.pallas import tpu as pltpu
from jax.experimental.pallas import tpu_sc as plsc

_NUM_CORES = 2
_NUM_SUBCORES = 16
_NW = _NUM_CORES * _NUM_SUBCORES
_CH = 8
_NBUF = 4
_A = 2


def _sc_copy_body(n_iter, x_hbm, o_hbm, sbuf_all, in_sems, out_sems):
    c = lax.axis_index("c")
    s = lax.axis_index("s")
    rows_per_w = n_iter * _CH
    base = (s * _NUM_CORES + c) * rows_per_w
    buf = sbuf_all.at[s]

    def in_copy(k, b):
        return pltpu.make_async_copy(
            x_hbm.at[pl.ds(base + k * _CH, _CH)], buf.at[b], in_sems.at[b]
        )

    def out_copy(k, b):
        return pltpu.make_async_copy(
            buf.at[b], o_hbm.at[pl.ds(base + k * _CH, _CH)], out_sems.at[b]
        )

    def body(g):
        for b in range(_NBUF):
            k = g * _NBUF + b

            @pl.when(k >= _NBUF)
            def _():
                out_copy(k - _NBUF, b).wait()

            in_copy(k, b).start()

            bb = (b - _A) % _NBUF

            @pl.when(k >= _A)
            def _():
                in_copy(k - _A, bb).wait()
                out_copy(k - _A, bb).start()

    pl.loop(0, n_iter // _NBUF)(body)
    for k in range(n_iter - _A, n_iter):
        in_copy(k, k % _NBUF).wait()
        out_copy(k, k % _NBUF).start()
    for k in range(n_iter - _NBUF, n_iter):
        out_copy(k, k % _NBUF).wait()


def kernel(inputs, memories):
    del memories
    B, T, d = inputs.shape
    rows = B * T
    n_iter = rows // (_NW * _CH)
    x = inputs.reshape(rows, d)
    mesh = plsc.VectorSubcoreMesh(core_axis_name="c", subcore_axis_name="s")
    run = functools.partial(
        pl.kernel,
        mesh=mesh,
        out_type=jax.ShapeDtypeStruct((rows, d), jnp.float32),
        scratch_types=[
            pltpu.VMEM_SHARED((_NUM_SUBCORES, _NBUF, _CH, d), jnp.float32),
            pltpu.SemaphoreType.DMA((_NBUF,)),
            pltpu.SemaphoreType.DMA((_NBUF,)),
        ],
    )(functools.partial(_sc_copy_body, n_iter))
    return run(x).reshape(B, T, d)
